# baseline (speedup 1.0000x reference)
import jax
import jax.numpy as jnp
from jax import lax
from jax.experimental import pallas as pl

N_TOKENS = 8192
D_MODEL = 2048
NUM_EXPERTS = 16
TOP_K = 2

BLOCK_T = 2048


def _router_body(x_ref, w_ref, b_ref, eps_ref, out_ref, idx_ref):
    xb = x_ref[...]
    both = jnp.dot(xb, w_ref[...], preferred_element_type=jnp.float32)
    both = both + b_ref[...]
    logits = both[:, :NUM_EXPERTS]
    nlogits = both[:, NUM_EXPERTS:]
    noisy = logits + eps_ref[...] * jax.nn.softplus(nlogits)

    iota = lax.broadcasted_iota(jnp.int32, noisy.shape, 1)
    m1 = jnp.max(noisy, axis=1, keepdims=True)
    i1 = jnp.min(jnp.where(noisy == m1, iota, NUM_EXPERTS), axis=1,
                 keepdims=True)
    masked = jnp.where(iota == i1, -jnp.inf, noisy)
    m2 = jnp.max(masked, axis=1, keepdims=True)
    i2 = jnp.min(jnp.where(masked == m2, iota, NUM_EXPERTS), axis=1,
                 keepdims=True)
    keep = (iota == i1) | (iota == i2)
    z = jnp.where(keep, jnp.exp(noisy - m1), 0.0)
    out_ref[...] = z / jnp.sum(z, axis=1, keepdims=True)
    idx_ref[...] = jnp.concatenate([i1, i2], axis=1)


def kernel(x, W_route, b_route, W_noise, b_noise, eps):
    n_blocks = N_TOKENS // BLOCK_T
    w_cat = jnp.concatenate([W_route, W_noise], axis=1)
    b_cat = jnp.concatenate([b_route, b_noise]).reshape(1, 2 * NUM_EXPERTS)
    out_shapes = (
        jax.ShapeDtypeStruct((N_TOKENS, NUM_EXPERTS), jnp.float32),
        jax.ShapeDtypeStruct((N_TOKENS, TOP_K), jnp.int32),
    )
    router_output, topk_indices = pl.pallas_call(
        _router_body,
        grid=(n_blocks,),
        in_specs=[
            pl.BlockSpec((BLOCK_T, D_MODEL), lambda i: (i, 0)),
            pl.BlockSpec((D_MODEL, 2 * NUM_EXPERTS), lambda i: (0, 0)),
            pl.BlockSpec((1, 2 * NUM_EXPERTS), lambda i: (0, 0)),
            pl.BlockSpec((BLOCK_T, NUM_EXPERTS), lambda i: (i, 0)),
        ],
        out_specs=(
            pl.BlockSpec((BLOCK_T, NUM_EXPERTS), lambda i: (i, 0)),
            pl.BlockSpec((BLOCK_T, TOP_K), lambda i: (i, 0)),
        ),
        out_shape=out_shapes,
    )(x, w_cat, b_cat, eps)
    return (router_output, topk_indices)


# speedup vs baseline: 1.0772x; 1.0772x over previous
import jax
import jax.numpy as jnp
from jax import lax
from jax.experimental import pallas as pl

N_TOKENS = 8192
D_MODEL = 2048
NUM_EXPERTS = 16
TOP_K = 2

BLOCK_T = 1024
HALF = BLOCK_T // 2


def _route_half(xb, w, b, epsb):
    both = jnp.dot(xb, w, preferred_element_type=jnp.float32) + b
    logits = both[:, :NUM_EXPERTS]
    nlogits = both[:, NUM_EXPERTS:]
    noisy = logits + epsb * jax.nn.softplus(nlogits)

    iota = lax.broadcasted_iota(jnp.int32, noisy.shape, 1)
    m1 = jnp.max(noisy, axis=1, keepdims=True)
    i1 = jnp.min(jnp.where(noisy == m1, iota, NUM_EXPERTS), axis=1,
                 keepdims=True)
    masked = jnp.where(iota == i1, -jnp.inf, noisy)
    m2 = jnp.max(masked, axis=1, keepdims=True)
    i2 = jnp.min(jnp.where(masked == m2, iota, NUM_EXPERTS), axis=1,
                 keepdims=True)
    keep = (iota == i1) | (iota == i2)
    z = jnp.where(keep, jnp.exp(noisy - m1), 0.0)
    out = z / jnp.sum(z, axis=1, keepdims=True)
    idx = jnp.concatenate([i1, i2], axis=1)
    return out, idx


def _router_body(x1_ref, x2_ref, w_ref, b_ref, eps_ref, out_ref, idx_ref):
    w = w_ref[...]
    b = b_ref[...]
    out1, idx1 = _route_half(x1_ref[...], w, b, eps_ref[:HALF, :])
    out_ref[:HALF, :] = out1
    idx_ref[:HALF, :] = idx1
    out2, idx2 = _route_half(x2_ref[...], w, b, eps_ref[HALF:, :])
    out_ref[HALF:, :] = out2
    idx_ref[HALF:, :] = idx2


def kernel(x, W_route, b_route, W_noise, b_noise, eps):
    n_blocks = N_TOKENS // BLOCK_T
    w_cat = jnp.concatenate([W_route, W_noise], axis=1)
    b_cat = jnp.concatenate([b_route, b_noise]).reshape(1, 2 * NUM_EXPERTS)
    out_shapes = (
        jax.ShapeDtypeStruct((N_TOKENS, NUM_EXPERTS), jnp.float32),
        jax.ShapeDtypeStruct((N_TOKENS, TOP_K), jnp.int32),
    )
    router_output, topk_indices = pl.pallas_call(
        _router_body,
        grid=(n_blocks,),
        in_specs=[
            pl.BlockSpec((HALF, D_MODEL), lambda i: (2 * i, 0)),
            pl.BlockSpec((HALF, D_MODEL), lambda i: (2 * i + 1, 0)),
            pl.BlockSpec((D_MODEL, 2 * NUM_EXPERTS), lambda i: (0, 0)),
            pl.BlockSpec((1, 2 * NUM_EXPERTS), lambda i: (0, 0)),
            pl.BlockSpec((BLOCK_T, NUM_EXPERTS), lambda i: (i, 0)),
        ],
        out_specs=(
            pl.BlockSpec((BLOCK_T, NUM_EXPERTS), lambda i: (i, 0)),
            pl.BlockSpec((BLOCK_T, TOP_K), lambda i: (i, 0)),
        ),
        out_shape=out_shapes,
    )(x, x, w_cat, b_cat, eps)
    return (router_output, topk_indices)


# speedup vs baseline: 1.0887x; 1.0107x over previous
import jax
import jax.numpy as jnp
from jax import lax
from jax.experimental import pallas as pl

N_TOKENS = 8192
D_MODEL = 2048
NUM_EXPERTS = 16
TOP_K = 2

N_STREAMS = 4
BLOCK_T = 2048
HALF = BLOCK_T // N_STREAMS


def _route_half(xb, w, b, epsb):
    both = jnp.dot(xb, w, preferred_element_type=jnp.float32) + b
    logits = both[:, :NUM_EXPERTS]
    nlogits = both[:, NUM_EXPERTS:]
    noisy = logits + epsb * jax.nn.softplus(nlogits)

    iota = lax.broadcasted_iota(jnp.int32, noisy.shape, 1)
    m1 = jnp.max(noisy, axis=1, keepdims=True)
    i1 = jnp.min(jnp.where(noisy == m1, iota, NUM_EXPERTS), axis=1,
                 keepdims=True)
    masked = jnp.where(iota == i1, -jnp.inf, noisy)
    m2 = jnp.max(masked, axis=1, keepdims=True)
    i2 = jnp.min(jnp.where(masked == m2, iota, NUM_EXPERTS), axis=1,
                 keepdims=True)
    keep = (iota == i1) | (iota == i2)
    z = jnp.where(keep, jnp.exp(noisy - m1), 0.0)
    out = z / jnp.sum(z, axis=1, keepdims=True)
    idx = jnp.concatenate([i1, i2], axis=1)
    return out, idx


def _router_body(*refs):
    x_refs = refs[:N_STREAMS]
    w_ref, b_ref, eps_ref, out_ref, idx_ref = refs[N_STREAMS:]
    w = w_ref[...]
    b = b_ref[...]
    for s in range(N_STREAMS):
        lo = s * HALF
        out_s, idx_s = _route_half(x_refs[s][...], w, b,
                                   eps_ref[pl.ds(lo, HALF), :])
        out_ref[pl.ds(lo, HALF), :] = out_s
        idx_ref[pl.ds(lo, HALF), :] = idx_s


def _x_spec(s):
    return pl.BlockSpec((HALF, D_MODEL), lambda i: (N_STREAMS * i + s, 0))


def kernel(x, W_route, b_route, W_noise, b_noise, eps):
    n_blocks = N_TOKENS // BLOCK_T
    w_cat = jnp.concatenate([W_route, W_noise], axis=1)
    b_cat = jnp.concatenate([b_route, b_noise]).reshape(1, 2 * NUM_EXPERTS)
    out_shapes = (
        jax.ShapeDtypeStruct((N_TOKENS, NUM_EXPERTS), jnp.float32),
        jax.ShapeDtypeStruct((N_TOKENS, TOP_K), jnp.int32),
    )
    router_output, topk_indices = pl.pallas_call(
        _router_body,
        grid=(n_blocks,),
        in_specs=[_x_spec(s) for s in range(N_STREAMS)] + [
            pl.BlockSpec((D_MODEL, 2 * NUM_EXPERTS), lambda i: (0, 0)),
            pl.BlockSpec((1, 2 * NUM_EXPERTS), lambda i: (0, 0)),
            pl.BlockSpec((BLOCK_T, NUM_EXPERTS), lambda i: (i, 0)),
        ],
        out_specs=(
            pl.BlockSpec((BLOCK_T, NUM_EXPERTS), lambda i: (i, 0)),
            pl.BlockSpec((BLOCK_T, TOP_K), lambda i: (i, 0)),
        ),
        out_shape=out_shapes,
    )(*([x] * N_STREAMS), w_cat, b_cat, eps)
    return (router_output, topk_indices)
